# baseline (speedup 1.0000x reference)
import jax
import jax.numpy as jnp
from jax.experimental import pallas as pl
from jax.experimental.pallas import tpu as pltpu


def _lrelu(x, slope):
    return jnp.maximum(x, slope * x)


def _layer(x, mj, addmask, scale_base, validf, tmT, Wf3, Wal, WarT, wae, We2, b2):
    H, _, O = Wf3.shape
    N = x.shape[0]
    xj = x[:mj, :]
    xT = x.T
    elT = jnp.dot(Wal, xT[:, :mj], preferred_element_type=jnp.float32)
    er = jnp.dot(x, WarT, preferred_element_type=jnp.float32)
    acc = jnp.zeros((N, O), jnp.float32)
    for h in range(H):
        ftj = jnp.dot(xj, Wf3[h], preferred_element_type=jnp.float32)
        e = elT[h : h + 1, :] + er[:, h : h + 1] + tmT * wae[h, 0]
        em = _lrelu(e, 0.2) + addmask
        emax = jnp.max(em, axis=1, keepdims=True)
        ex = jnp.exp(em - emax)
        esum = jnp.sum(ex, axis=1, keepdims=True)
        alpha = ex * (validf / (esum + scale_base))
        out1 = jnp.dot(alpha, ftj, preferred_element_type=jnp.float32)
        s = jnp.sum(alpha * tmT, axis=1, keepdims=True)
        y = out1 + s * We2[h : h + 1, :] + b2[h : h + 1, :]
        acc = acc + _lrelu(y, 0.01)
    return acc * (1.0 / H)


def _gnn_body(nf_ref, g_ref, t_ref,
              Wf3_0, Wal_0, WarT_0, wae_0, We2_0, b2_0,
              Wf3_1, Wal_1, WarT_1, wae_1, We2_1, b2_1,
              Wf3_2, Wal_2, WarT_2, wae_2, We2_2, b2_2,
              out_ref):
    mj = g_ref.shape[1]
    N = g_ref.shape[2]
    x0 = nf_ref[0]
    gT = g_ref[0].T
    addmask = (gT.astype(jnp.float32) - 1.0) * 1e30
    indeg = jnp.sum(gT.astype(jnp.float32), axis=1, keepdims=True)
    validf = jnp.minimum(indeg, 1.0)
    scale_base = 1.0 - validf
    tT = t_ref[0].T
    tmT = jnp.concatenate(
        [tT, jnp.zeros((N - mj, mj), jnp.float32)], axis=0)

    x1 = _layer(x0, mj, addmask, scale_base, validf, tmT,
                Wf3_0[:], Wal_0[:], WarT_0[:], wae_0[:], We2_0[:], b2_0[:])
    x2 = _layer(x1, mj, addmask, scale_base, validf, tmT,
                Wf3_1[:], Wal_1[:], WarT_1[:], wae_1[:], We2_1[:], b2_1[:])
    x3 = _layer(x2, mj, addmask, scale_base, validf, tmT,
                Wf3_2[:], Wal_2[:], WarT_2[:], wae_2[:], We2_2[:], b2_2[:])
    out_ref[0] = x3


def _prep_weights(Wf, We, al, ar, ae, b):
    H, O = al.shape
    fi = Wf.shape[0]
    Wf3 = Wf.reshape(fi, H, O).transpose(1, 0, 2)
    Wal = jnp.einsum("hio,ho->hi", Wf3, al)
    WarT = jnp.einsum("hio,ho->hi", Wf3, ar).T
    We2 = We.reshape(H, O)
    wae = jnp.sum(We2 * ae, axis=1, keepdims=True)
    b2 = b.reshape(H, O)
    return Wf3, Wal, WarT, wae, We2, b2


def kernel(Graph, norm_h, norm_L, norm_W, norm_P, norm_N, numberOfJobs,
           numberOfMachines, T, Wf0, We0, al0, ar0, ae0, b0,
           Wf1, We1, al1, ar1, ae1, b1, Wf2, We2, al2, ar2, ae2, b2):
    bs, mj, N = Graph.shape
    mm = N - mj
    H, Ofin = al2.shape

    f32 = jnp.float32
    jmask = (jnp.arange(mj)[None, :] < numberOfJobs).astype(f32)
    mmask = (jnp.arange(mm)[None, :] < numberOfMachines).astype(f32)
    jobID = jnp.arange(1, mj + 1, dtype=f32)[None, :] * jmask
    machID = jnp.arange(1, mm + 1, dtype=f32)[None, :] * mmask
    jz = jnp.zeros((bs, mj), f32)
    jobF = jnp.stack([norm_h, norm_L, jz, jz, jz, jobID, jz], axis=-1)
    mzc = jnp.zeros((bs, mm), f32)
    Wb = jnp.broadcast_to(norm_W, (bs, mm))
    Pb = jnp.broadcast_to(norm_P, (bs, mm))
    Nb = jnp.broadcast_to(norm_N, (bs, mm))
    machF = jnp.stack([mzc, mzc, Wb, Pb, Nb, mzc, machID], axis=-1)
    nf = jnp.concatenate([jobF, machF], axis=1)

    w0 = _prep_weights(Wf0, We0, al0, ar0, ae0, b0)
    w1 = _prep_weights(Wf1, We1, al1, ar1, ae1, b1)
    w2 = _prep_weights(Wf2, We2, al2, ar2, ae2, b2)

    def batch_spec(*dims):
        return pl.BlockSpec((1,) + dims, lambda bb: (bb, 0, 0))

    def full_spec(arr):
        nd = arr.ndim
        return pl.BlockSpec(arr.shape, lambda bb: (0,) * nd)

    weight_ops = list(w0) + list(w1) + list(w2)
    in_specs = ([batch_spec(N, 7), batch_spec(mj, N), batch_spec(mj, mj)]
                + [full_spec(a) for a in weight_ops])

    out = pl.pallas_call(
        _gnn_body,
        grid=(bs,),
        in_specs=in_specs,
        out_specs=pl.BlockSpec((1, N, Ofin), lambda bb: (bb, 0, 0)),
        out_shape=jax.ShapeDtypeStruct((bs, N, Ofin), f32),
        compiler_params=pltpu.CompilerParams(
            dimension_semantics=("parallel",)),
    )(nf, Graph, T, *weight_ops)
    return out
